# baseline (speedup 1.0000x reference)
import jax
import jax.numpy as jnp
from jax.experimental import pallas as pl
from jax.experimental.pallas import tpu as pltpu

_B, _S, _D, _F, _E = 4, 2048, 1024, 4096, 8
_BF = 1024


def _gate_kernel(h_ref, wg_ref, logits_ref, choice_ref):
    hbar = jnp.mean(h_ref[...], axis=1)
    logits = jnp.dot(hbar, wg_ref[...], preferred_element_type=jnp.float32)
    logits_ref[...] = logits
    row_max = jnp.max(logits, axis=-1, keepdims=True)
    idx = jax.lax.broadcasted_iota(jnp.int32, logits.shape, 1)
    masked = jnp.where(logits == row_max, idx, _E)
    choice_ref[...] = jnp.min(masked, axis=-1, keepdims=True)


def _moe_kernel(choice_ref, h_ref, w1_ref, w2_ref, out_ref):
    fj = pl.program_id(1)
    contrib = jnp.dot(
        jax.nn.gelu(
            jnp.dot(h_ref[0], w1_ref[0], preferred_element_type=jnp.float32)
        ),
        w2_ref[0],
        preferred_element_type=jnp.float32,
    )

    @pl.when(fj == 0)
    def _first():
        out_ref[0, :, :] = contrib

    @pl.when(fj > 0)
    def _rest():
        out_ref[0, :, :] += contrib


def kernel(hidden_states, Wg, W1, W2):
    logits, choice = pl.pallas_call(
        _gate_kernel,
        out_shape=(
            jax.ShapeDtypeStruct((_B, _E), jnp.float32),
            jax.ShapeDtypeStruct((_B, 1), jnp.int32),
        ),
    )(hidden_states, Wg)

    choice_1d = choice.reshape(_B)

    grid_spec = pltpu.PrefetchScalarGridSpec(
        num_scalar_prefetch=1,
        grid=(_B, _F // _BF),
        in_specs=[
            pl.BlockSpec((1, _S, _D), lambda b, j, c: (b, 0, 0)),
            pl.BlockSpec((1, _D, _BF), lambda b, j, c: (c[b], 0, j)),
            pl.BlockSpec((1, _BF, _D), lambda b, j, c: (c[b], j, 0)),
        ],
        out_specs=pl.BlockSpec((1, _S, _D), lambda b, j, c: (b, 0, 0)),
    )
    out = pl.pallas_call(
        _moe_kernel,
        grid_spec=grid_spec,
        out_shape=jax.ShapeDtypeStruct((_B, _S, _D), jnp.float32),
        compiler_params=pltpu.CompilerParams(
            dimension_semantics=("arbitrary", "arbitrary"),
            vmem_limit_bytes=100 * 1024 * 1024,
        ),
    )(choice_1d, hidden_states, W1, W2)

    return (out, logits)


# speedup vs baseline: 1.1326x; 1.1326x over previous
import jax
import jax.numpy as jnp
from jax.experimental import pallas as pl
from jax.experimental.pallas import tpu as pltpu

_B, _S, _D, _F, _E = 4, 2048, 1024, 4096, 8
_BF = 1024
_RS = 512


def _gate_kernel(h_ref, wg_ref, logits_ref, choice_ref):
    hbar = jnp.mean(h_ref[...], axis=1)
    logits = jnp.dot(hbar, wg_ref[...], preferred_element_type=jnp.float32)
    logits_ref[...] = logits
    row_max = jnp.max(logits, axis=-1, keepdims=True)
    idx = jax.lax.broadcasted_iota(jnp.int32, logits.shape, 1)
    masked = jnp.where(logits == row_max, idx, _E)
    choice_ref[...] = jnp.min(masked, axis=-1, keepdims=True)


def _moe_kernel(choice_ref, h_ref, w1_ref, w2_ref, out_ref):
    fj = pl.program_id(1)

    @pl.when(fj == 0)
    def _init():
        out_ref[...] = jnp.zeros_like(out_ref)

    w1 = w1_ref[0]
    w2 = w2_ref[0]
    for r in range(0, _S, _RS):
        hmid = jax.nn.gelu(
            jnp.dot(
                h_ref[0, r : r + _RS, :], w1,
                preferred_element_type=jnp.float32,
            )
        )
        out_ref[0, r : r + _RS, :] += jnp.dot(
            hmid, w2, preferred_element_type=jnp.float32
        )


def kernel(hidden_states, Wg, W1, W2):
    logits, choice = pl.pallas_call(
        _gate_kernel,
        out_shape=(
            jax.ShapeDtypeStruct((_B, _E), jnp.float32),
            jax.ShapeDtypeStruct((_B, 1), jnp.int32),
        ),
    )(hidden_states, Wg)

    choice_1d = choice.reshape(_B)

    grid_spec = pltpu.PrefetchScalarGridSpec(
        num_scalar_prefetch=1,
        grid=(_B, _F // _BF),
        in_specs=[
            pl.BlockSpec((1, _S, _D), lambda b, j, c: (b, 0, 0)),
            pl.BlockSpec((1, _D, _BF), lambda b, j, c: (c[b], 0, j)),
            pl.BlockSpec((1, _BF, _D), lambda b, j, c: (c[b], j, 0)),
        ],
        out_specs=pl.BlockSpec((1, _S, _D), lambda b, j, c: (b, 0, 0)),
    )
    out = pl.pallas_call(
        _moe_kernel,
        grid_spec=grid_spec,
        out_shape=jax.ShapeDtypeStruct((_B, _S, _D), jnp.float32),
        compiler_params=pltpu.CompilerParams(
            dimension_semantics=("arbitrary", "arbitrary"),
            vmem_limit_bytes=100 * 1024 * 1024,
        ),
    )(choice_1d, hidden_states, W1, W2)

    return (out, logits)


# speedup vs baseline: 1.1388x; 1.0055x over previous
import jax
import jax.numpy as jnp
from jax.experimental import pallas as pl
from jax.experimental.pallas import tpu as pltpu

_B, _S, _D, _F, _E = 4, 2048, 1024, 4096, 8
_BF = 1024
_RS = 512


def _gate_kernel(h_ref, wg_ref, logits_ref, choice_ref):
    hbar = jnp.mean(h_ref[...], axis=1)
    logits = jnp.dot(hbar, wg_ref[...], preferred_element_type=jnp.float32)
    logits_ref[...] = logits
    row_max = jnp.max(logits, axis=-1, keepdims=True)
    idx = jax.lax.broadcasted_iota(jnp.int32, logits.shape, 1)
    masked = jnp.where(logits == row_max, idx, _E)
    choice_ref[...] = jnp.min(masked, axis=-1, keepdims=True)


_NF = _F // _BF


def _moe_kernel(choice_ref, h_ref, w1_hbm, w2_hbm, out_ref, w1buf, w2buf, sems):
    b = pl.program_id(0)
    c = choice_ref[b]
    c_next = choice_ref[jnp.minimum(b + 1, _B - 1)]

    def w1_copy(expert, fj, slot):
        return pltpu.make_async_copy(
            w1_hbm.at[expert, :, pl.ds(fj * _BF, _BF)],
            w1buf.at[slot],
            sems.at[0, slot],
        )

    def w2_copy(expert, fj, slot):
        return pltpu.make_async_copy(
            w2_hbm.at[expert, pl.ds(fj * _BF, _BF), :],
            w2buf.at[slot],
            sems.at[1, slot],
        )

    def issue(expert, fj, slot):
        w1_copy(expert, fj, slot).start()
        w2_copy(expert, fj, slot).start()

    @pl.when(b == 0)
    def _prologue():
        issue(c, 0, 0)
        issue(c, 1, 1)

    for fj in range(_NF):
        slot = fj % 2
        w1_copy(c, fj, slot).wait()
        w2_copy(c, fj, slot).wait()
        w1 = w1buf[slot]
        w2 = w2buf[slot]
        for r in range(0, _S, _RS):
            hmid = jax.nn.gelu(
                jnp.dot(
                    h_ref[0, r : r + _RS, :], w1,
                    preferred_element_type=jnp.float32,
                )
            )
            contrib = jnp.dot(hmid, w2, preferred_element_type=jnp.float32)
            if fj == 0:
                out_ref[0, r : r + _RS, :] = contrib
            else:
                out_ref[0, r : r + _RS, :] += contrib

        nfj = fj + 2
        if nfj < _NF:
            issue(c, nfj, nfj % 2)
        else:

            @pl.when(b + 1 < _B)
            def _issue_next_b():
                issue(c_next, nfj - _NF, nfj % 2)


def kernel(hidden_states, Wg, W1, W2):
    logits, choice = pl.pallas_call(
        _gate_kernel,
        out_shape=(
            jax.ShapeDtypeStruct((_B, _E), jnp.float32),
            jax.ShapeDtypeStruct((_B, 1), jnp.int32),
        ),
    )(hidden_states, Wg)

    choice_1d = choice.reshape(_B)

    grid_spec = pltpu.PrefetchScalarGridSpec(
        num_scalar_prefetch=1,
        grid=(_B,),
        in_specs=[
            pl.BlockSpec((1, _S, _D), lambda b, c: (b, 0, 0)),
            pl.BlockSpec(memory_space=pl.ANY),
            pl.BlockSpec(memory_space=pl.ANY),
        ],
        out_specs=pl.BlockSpec((1, _S, _D), lambda b, c: (b, 0, 0)),
        scratch_shapes=[
            pltpu.VMEM((2, _D, _BF), jnp.float32),
            pltpu.VMEM((2, _BF, _D), jnp.float32),
            pltpu.SemaphoreType.DMA((2, 2)),
        ],
    )
    out = pl.pallas_call(
        _moe_kernel,
        grid_spec=grid_spec,
        out_shape=jax.ShapeDtypeStruct((_B, _S, _D), jnp.float32),
        compiler_params=pltpu.CompilerParams(
            dimension_semantics=("arbitrary",),
            vmem_limit_bytes=100 * 1024 * 1024,
        ),
    )(choice_1d, hidden_states, W1, W2)

    return (out, logits)
